# Initial kernel scaffold; baseline (speedup 1.0000x reference)
import jax
import jax.numpy as jnp
from jax.experimental import pallas as pl


def kernel(x, edge_index, edge_mask_batch, prep_W1, prep_b1, prep_W2, prep_b2, prep_W3, prep_b3, msg_W1, msg_b1, msg_W2, msg_b2, msg_W3, msg_b3, upd_W1, upd_b1, upd_W2, upd_b2, upd_W3, upd_b3):
    raise NotImplementedError("write your pallas kernel here")


# speedup vs baseline: 8.9348x; 8.9348x over previous
import functools

import jax
import jax.numpy as jnp
from jax import lax
from jax.experimental import pallas as pl
from jax.experimental.pallas import tpu as pltpu
from jax.experimental.pallas import tpu_sc as plsc

N = 10000
E = 160000
DIM = 256
DEPTH = 8

NC, NS = 2, 16
NW = NC * NS
CH = 128
NCHUNK = 40
EPT = CH * NCHUNK
EPAD = EPT * NW
NPAD = 10240
RPT = NPAD // NS
TRASH = N

_HI = lax.Precision.HIGHEST


def _sc_segment_sum(aug, srcp, dstp, maskp, zeros):
    mesh = plsc.VectorSubcoreMesh(core_axis_name="c", subcore_axis_name="s")

    @functools.partial(
        pl.kernel,
        out_type=jax.ShapeDtypeStruct((NC, NPAD, 16), jnp.float32),
        mesh=mesh,
        compiler_params=pltpu.CompilerParams(use_tc_tiling_on_sc=False),
        scratch_types=[
            pltpu.VMEM((NCHUNK, CH), jnp.int32),
            pltpu.VMEM((NCHUNK, CH), jnp.int32),
            pltpu.VMEM((NCHUNK, CH), jnp.int32),
            pltpu.VMEM((2, CH, 16), jnp.float32),
            pltpu.VMEM((RPT, 16), jnp.float32),
            pltpu.VMEM_SHARED((NPAD, 16), jnp.float32),
            pltpu.SemaphoreType.DMA((2,)),
        ],
    )
    def k(aug_hbm, src_hbm, dst_hbm, msk_hbm, zero_hbm, out_hbm,
          idx_s, idx_d, idx_m, gbuf, zbuf, acc, gsem):
        cid = lax.axis_index("c")
        sid = lax.axis_index("s")
        tid = cid * NS + sid

        pltpu.sync_copy(src_hbm.at[tid], idx_s)
        pltpu.sync_copy(dst_hbm.at[tid], idx_d)
        pltpu.sync_copy(msk_hbm.at[tid], idx_m)

        pltpu.sync_copy(zero_hbm, zbuf)
        pltpu.sync_copy(zbuf, acc.at[pl.ds(sid * RPT, RPT)])

        trash = jnp.full((16,), TRASH, jnp.int32)

        def mask_body(j, _):
            for k16 in range(CH // 16):
                sl = pl.ds(k16 * 16, 16)
                m = idx_m[j, sl]
                d = idx_d[j, sl]
                idx_d[j, sl] = jnp.where(m != 0, d, trash)
            return 0

        lax.fori_loop(0, NCHUNK, mask_body, 0)
        plsc.subcore_barrier()

        pltpu.async_copy(aug_hbm.at[idx_s.at[0]], gbuf.at[0], gsem.at[0])
        pltpu.async_copy(aug_hbm.at[idx_s.at[1]], gbuf.at[1], gsem.at[1])

        def chunk_body(j0, _):
            for b in range(2):
                j = j0 * 2 + b
                pltpu.make_async_copy(aug_hbm.at[idx_s.at[j]],
                                      gbuf.at[b], gsem.at[b]).wait()
                pltpu.sync_copy(gbuf.at[b], acc.at[idx_d.at[j]], add=True)

                @pl.when(j + 2 < NCHUNK)
                def _():
                    pltpu.async_copy(aug_hbm.at[idx_s.at[j + 2]],
                                     gbuf.at[b], gsem.at[b])
            return 0

        lax.fori_loop(0, NCHUNK // 2, chunk_body, 0)
        plsc.subcore_barrier()

        pltpu.sync_copy(acc.at[pl.ds(sid * RPT, RPT)], zbuf)
        pltpu.sync_copy(zbuf, out_hbm.at[cid, pl.ds(sid * RPT, RPT)])

    return k(aug, srcp, dstp, maskp, zeros)


_B = 2000
_G = N // _B


def _full(shape):
    return pl.BlockSpec(shape, lambda i: tuple(0 for _ in shape))


def _dot(a, b):
    return jnp.dot(a, b, preferred_element_type=jnp.float32, precision=_HI)


def _aug_from_h(h, mW1, mb1, mW2p, mb2p, e8):
    a1 = jnp.maximum(_dot(h, mW1) + mb1, 0.0)
    return jnp.maximum(_dot(a1, mW2p) + mb2p, 0.0) + e8


def _tc_prep(xp, pW1p, pb1, pW2, pb2, pW3, pb3, mW1, mb1, mW2p, mb2p, e8):
    def body(x_ref, w1, b1, w2, b2, w3, b3, mw1, mb1_, mw2, mb2_, e8_,
             h_ref, aug_ref):
        h1 = jnp.maximum(_dot(x_ref[...], w1[...]) + b1[...], 0.0)
        h2 = jnp.maximum(_dot(h1, w2[...]) + b2[...], 0.0)
        h = _dot(h2, w3[...]) + b3[...]
        h_ref[...] = h
        aug_ref[...] = _aug_from_h(h, mw1[...], mb1_[...], mw2[...],
                                   mb2_[...], e8_[...])

    return pl.pallas_call(
        body,
        grid=(_G,),
        in_specs=[
            pl.BlockSpec((_B, 8), lambda i: (i, 0)),
            _full((8, 16)), _full((1, 16)),
            _full((16, 8)), _full((1, 8)),
            _full((8, DIM)), _full((1, DIM)),
            _full((DIM, 16)), _full((1, 16)),
            _full((16, 16)), _full((1, 16)),
            _full((1, 16)),
        ],
        out_specs=[
            pl.BlockSpec((_B, DIM), lambda i: (i, 0)),
            pl.BlockSpec((_B, 16), lambda i: (i, 0)),
        ],
        out_shape=[
            jax.ShapeDtypeStruct((N, DIM), jnp.float32),
            jax.ShapeDtypeStruct((N, 16), jnp.float32),
        ],
    )(xp, pW1p, pb1, pW2, pb2, pW3, pb3, mW1, mb1, mW2p, mb2p, e8)


def _tc_step(h, pa, pb, mW3, mb3, uW1, ub1, uW2p, ub2p, uW3p, ub3,
             mW1, mb1, mW2p, mb2p, e8):
    def body(h_ref, pa_ref, pb_ref, mw3, mb3_, uw1, ub1_, uw2, ub2_,
             uw3, ub3_, mw1, mb1_, mw2, mb2_, e8_, hn_ref, aug_ref):
        u = pa_ref[...] + pb_ref[...]
        s8 = u[:, :8]
        cnt = u[:, 8:9]
        wc = _dot(mw3[...], uw1[...])
        bc = _dot(mb3_[...], uw1[...])
        u1 = jnp.maximum(_dot(s8, wc) + cnt * bc + ub1_[...], 0.0)
        u8 = jnp.maximum(_dot(u1, uw2[...]) + ub2_[...], 0.0)
        d = _dot(u8, uw3[...])
        hn = h_ref[...] + jnp.where(cnt > 0.0, d + ub3_[...], 0.0)
        hn_ref[...] = hn
        aug_ref[...] = _aug_from_h(hn, mw1[...], mb1_[...], mw2[...],
                                   mb2_[...], e8_[...])

    return pl.pallas_call(
        body,
        grid=(_G,),
        in_specs=[
            pl.BlockSpec((_B, DIM), lambda i: (i, 0)),
            pl.BlockSpec((_B, 16), lambda i: (i, 0)),
            pl.BlockSpec((_B, 16), lambda i: (i, 0)),
            _full((8, DIM)), _full((1, DIM)),
            _full((DIM, 16)), _full((1, 16)),
            _full((16, 16)), _full((1, 16)),
            _full((16, DIM)), _full((1, DIM)),
            _full((DIM, 16)), _full((1, 16)),
            _full((16, 16)), _full((1, 16)),
            _full((1, 16)),
        ],
        out_specs=[
            pl.BlockSpec((_B, DIM), lambda i: (i, 0)),
            pl.BlockSpec((_B, 16), lambda i: (i, 0)),
        ],
        out_shape=[
            jax.ShapeDtypeStruct((N, DIM), jnp.float32),
            jax.ShapeDtypeStruct((N, 16), jnp.float32),
        ],
    )(h, pa, pb, mW3, mb3, uW1, ub1, uW2p, ub2p, uW3p, ub3,
      mW1, mb1, mW2p, mb2p, e8)


def kernel(x, edge_index, edge_mask_batch,
           prep_W1, prep_b1, prep_W2, prep_b2, prep_W3, prep_b3,
           msg_W1, msg_b1, msg_W2, msg_b2, msg_W3, msg_b3,
           upd_W1, upd_b1, upd_W2, upd_b2, upd_W3, upd_b3):
    f32 = jnp.float32
    xp = jnp.pad(x, ((0, 0), (0, 3)))
    pW1p = jnp.pad(prep_W1, ((0, 3), (0, 0)))
    mW2p = jnp.pad(msg_W2, ((0, 0), (0, 8)))
    mb2p = jnp.pad(msg_b2, (0, 8)).reshape(1, 16)
    uW2p = jnp.pad(upd_W2, ((0, 0), (0, 8)))
    ub2p = jnp.pad(upd_b2, (0, 8)).reshape(1, 16)
    uW3p = jnp.pad(upd_W3, ((0, 8), (0, 0)))
    e8 = jnp.zeros((1, 16), f32).at[0, 8].set(1.0)

    pb1 = prep_b1.reshape(1, 16)
    pb2 = prep_b2.reshape(1, 8)
    pb3 = prep_b3.reshape(1, DIM)
    mb1 = msg_b1.reshape(1, 16)
    mb3 = msg_b3.reshape(1, DIM)
    ub1 = upd_b1.reshape(1, 16)
    ub3 = upd_b3.reshape(1, DIM)

    dst = edge_index[0]
    src = edge_index[1]
    srcp = jnp.pad(src, (0, EPAD - E)).reshape(NW, NCHUNK, CH)
    dstp = jnp.pad(dst, (0, EPAD - E)).reshape(NW, NCHUNK, CH)
    maskp = jnp.pad(edge_mask_batch.astype(jnp.int32),
                    ((0, 0), (0, EPAD - E))).reshape(DEPTH, NW, NCHUNK, CH)
    zeros = jnp.zeros((RPT, 16), f32)

    h, aug = _tc_prep(xp, pW1p, pb1, prep_W2, pb2, prep_W3, pb3,
                      msg_W1, mb1, mW2p, mb2p, e8)
    for d in range(DEPTH):
        parts = _sc_segment_sum(aug, srcp, dstp, maskp[d], zeros)
        h, aug = _tc_step(h, parts[0, :N], parts[1, :N],
                          msg_W3, mb3, upd_W1, ub1, uW2p, ub2p, uW3p, ub3,
                          msg_W1, mb1, mW2p, mb2p, e8)
    return h
